# baseline (speedup 1.0000x reference)
import functools

import jax
import jax.numpy as jnp
from jax import lax
from jax.experimental import pallas as pl
from jax.experimental.pallas import tpu as pltpu
from jax.experimental.pallas import tpu_sc as plsc

SEQ = 200
WORD_DIM = 64
HIDDEN = 128
L = 16
ROWS = 8
NGROUP = SEQ // ROWS


def _body(sent_hbm, wordt_hbm, post_hbm, w_hbm, b_hbm,
          out_hbm, hid2_hbm,
          sv_v, stage_v, posblk_v, outblk_v, psum_v, psums_v, wt_v,
          bvec_v, hidout_v, shared, *sems):
    c = lax.axis_index("c")
    s = lax.axis_index("s")
    gid = c * 16 + s
    active = gid < NGROUP
    base = gid * ROWS

    @pl.when(active)
    def _gather_group():
        pltpu.sync_copy(sent_hbm.at[pl.ds(base, ROWS)], sv_v.at[pl.ds(0, ROWS)])
        pltpu.sync_copy(post_hbm.at[pl.ds(base, ROWS)], posblk_v)
        sv = sv_v[...]
        qv = lax.shift_left(lax.shift_right_logical(sv, 7), 7)
        cv = sv & 127
        copies = []
        for r in range(ROWS):
            qr = pl.multiple_of(qv[r], HIDDEN)
            copies.append(pltpu.async_copy(
                wordt_hbm.at[:, pl.ds(qr, HIDDEN)],
                stage_v.at[pl.ds(r * WORD_DIM, WORD_DIM)], sems[r]))
        lane_iota = lax.iota(jnp.int32, L)
        accw = [jnp.zeros((L,), jnp.float32) for _ in range(4)]
        accp = [jnp.zeros((L,), jnp.float32) for _ in range(4)]
        for r in range(ROWS):
            copies[r].wait()
            cvec = jnp.full((L,), cv[r], jnp.int32)
            for ch in range(4):
                jv = lane_iota + (r * WORD_DIM + ch * L)
                wv = plsc.load_gather(stage_v, [jv, cvec])
                pv = posblk_v[r, pl.ds(ch * L, L)]
                outblk_v[r, pl.ds(ch * L, L)] = wv
                outblk_v[r, pl.ds(WORD_DIM + ch * L, L)] = pv
                accw[ch] = accw[ch] + wv
                accp[ch] = accp[ch] + pv
        pltpu.sync_copy(outblk_v, out_hbm.at[pl.ds(base, ROWS)])
        for ch in range(4):
            psum_v[0, pl.ds(ch * L, L)] = accw[ch]
            psum_v[0, pl.ds(WORD_DIM + ch * L, L)] = accp[ch]

    @pl.when(jnp.logical_not(active))
    def _zero_psum():
        z = jnp.zeros((L,), jnp.float32)
        for ch in range(8):
            psum_v[0, pl.ds(ch * L, L)] = z

    pltpu.sync_copy(psum_v, shared.at[pl.ds(s, 1)])
    plsc.subcore_barrier()

    @pl.when(s < 8)
    def _linear_phase():
        pltpu.sync_copy(shared, psums_v)
        pltpu.sync_copy(w_hbm.at[pl.ds(s * L, L)], wt_v)
        pltpu.sync_copy(b_hbm.at[pl.ds(s * L, L)], bvec_v)
        lane_iota = lax.iota(jnp.int32, L)
        totals = []
        for kc in range(8):
            vals = [psums_v[w, pl.ds(kc * L, L)] for w in range(16)]
            while len(vals) > 1:
                vals = [vals[i] + vals[i + 1] for i in range(0, len(vals), 2)]
            totals.append(vals[0] * (1.0 / SEQ))
        dnums = lax.GatherDimensionNumbers(
            offset_dims=(), collapsed_slice_dims=(0,), start_index_map=(0,))
        bvec = bvec_v[...]
        accs = [jnp.where(c == 0, bvec, jnp.zeros((L,), jnp.float32)),
                jnp.zeros((L,), jnp.float32),
                jnp.zeros((L,), jnp.float32),
                jnp.zeros((L,), jnp.float32)]
        for k in range(HIDDEN):
            lane = jnp.full((L, 1), k % L, jnp.int32)
            scal = lax.gather(totals[k // L], lane, dnums, (1,),
                              mode=lax.GatherScatterMode.PROMISE_IN_BOUNDS)
            wcol = plsc.load_gather(wt_v, [lane_iota,
                                           jnp.full((L,), k, jnp.int32)])
            accs[k % 4] = accs[k % 4] + scal * wcol
        hidout_v[...] = (accs[0] + accs[1]) + (accs[2] + accs[3])
        pltpu.sync_copy(hidout_v, hid2_hbm.at[pl.ds(c * HIDDEN + s * L, L)])


@jax.jit
def _encode(sent, wordt, post, w, b):
    mesh = plsc.VectorSubcoreMesh(core_axis_name="c", subcore_axis_name="s")
    run = functools.partial(
        pl.kernel,
        mesh=mesh,
        compiler_params=pltpu.CompilerParams(
            use_tc_tiling_on_sc=True, needs_layout_passes=False),
        out_type=[
            jax.ShapeDtypeStruct((SEQ, HIDDEN), jnp.float32),
            jax.ShapeDtypeStruct((2 * HIDDEN,), jnp.float32),
        ],
        scratch_types=[
            pltpu.VMEM((L,), jnp.int32),
            pltpu.VMEM((ROWS * WORD_DIM, HIDDEN), jnp.float32),
            pltpu.VMEM((ROWS, HIDDEN), jnp.float32),
            pltpu.VMEM((ROWS, HIDDEN), jnp.float32),
            pltpu.VMEM((1, HIDDEN), jnp.float32),
            pltpu.VMEM((16, HIDDEN), jnp.float32),
            pltpu.VMEM((L, HIDDEN), jnp.float32),
            pltpu.VMEM((L,), jnp.float32),
            pltpu.VMEM((L,), jnp.float32),
            pltpu.VMEM_SHARED((16, HIDDEN), jnp.float32),
        ] + [pltpu.SemaphoreType.DMA] * ROWS,
    )(_body)
    return run(sent, wordt, post, w, b)


def kernel(sentence, word_table, pos_table, W, b):
    sent = sentence.astype(jnp.int32)
    wordt = word_table.T
    post = jnp.pad(pos_table, ((0, 0), (0, HIDDEN - WORD_DIM)))
    out, hid2 = _encode(sent, wordt, post, W, b)
    hid = hid2.reshape(2, HIDDEN).sum(axis=0)
    return out.reshape(SEQ, 1, HIDDEN), hid.reshape(1, 1, HIDDEN)
